# Initial kernel scaffold; baseline (speedup 1.0000x reference)
import jax
import jax.numpy as jnp
from jax.experimental import pallas as pl


def kernel(x, edge_index, W1, b1, W2, b2):
    raise NotImplementedError("write your pallas kernel here")


# speedup vs baseline: 15.3069x; 15.3069x over previous
import functools

import jax
import jax.numpy as jnp
from jax import lax
from jax.experimental import pallas as pl
from jax.experimental.pallas import tpu as pltpu
from jax.experimental.pallas import tpu_sc as plsc

N = 10000
D = 128
H = 128
NP = 10240
TRASH = 10000
NC = 2
NS = 16
L = 16
NW = NC * NS
CH = 128
CPT = 80
EP = NW * CPT * CH
NCH = EP // CH
BLK = 1024
TG = NP // BLK
NPS = NP // NS

_mesh = plsc.VectorSubcoreMesh(
    core_axis_name="c", subcore_axis_name="s", num_cores=NC, num_subcores=NS
)


@functools.partial(
    pl.kernel,
    out_type=jax.ShapeDtypeStruct((NC, NP), jnp.float32),
    mesh=_mesh,
    scratch_types=[
        pltpu.VMEM((CPT, CH), jnp.int32),
        pltpu.VMEM((CH,), jnp.float32),
        pltpu.VMEM_SHARED((NP,), jnp.float32),
    ],
)
def _deg_kernel(dst2d, zeros_np, deg_out, didx, ones, deg_sp):
    cid = lax.axis_index("c")
    sid = lax.axis_index("s")
    wid = cid * NS + sid
    for j in range(CH // L):
        ones[pl.ds(j * L, L)] = jnp.ones((L,), jnp.float32)
    pltpu.sync_copy(dst2d.at[pl.ds(wid * CPT, CPT)], didx)

    @pl.when(sid == 0)
    def _():
        pltpu.sync_copy(zeros_np, deg_sp)

    plsc.subcore_barrier()

    def body(k, carry):
        pltpu.sync_copy(ones, deg_sp.at[didx.at[k]], add=True)
        return carry

    lax.fori_loop(0, CPT, body, 0)
    plsc.subcore_barrier()
    pltpu.sync_copy(
        deg_sp.at[pl.ds(sid * NPS, NPS)],
        deg_out.at[cid, pl.ds(sid * NPS, NPS)],
    )


GRP = 16
NG = CPT // GRP


@functools.partial(
    pl.kernel,
    out_type=jax.ShapeDtypeStruct((NC, NP, H), jnp.float32),
    mesh=_mesh,
    scratch_types=[
        pltpu.VMEM((GRP, CH), jnp.int32),
        pltpu.VMEM((GRP, CH), jnp.int32),
        pltpu.VMEM((2, CH, H), jnp.float32),
        pltpu.VMEM_SHARED((NP, H), jnp.float32),
        pltpu.SemaphoreType.DMA,
        pltpu.SemaphoreType.DMA,
    ],
)
def _agg_kernel(src2d, dst2d, u_hbm, zeros_nph, acc_out,
                sidx, didx, rows, acc_sp, sem0, sem1):
    cid = lax.axis_index("c")
    sid = lax.axis_index("s")
    wid = cid * NS + sid

    @pl.when(sid == 0)
    def _():
        pltpu.sync_copy(zeros_nph, acc_sp)

    plsc.subcore_barrier()

    def gbody(g, carry):
        base = wid * CPT + g * GRP
        pltpu.sync_copy(src2d.at[pl.ds(base, GRP)], sidx)
        pltpu.sync_copy(dst2d.at[pl.ds(base, GRP)], didx)
        pltpu.async_copy(u_hbm.at[sidx.at[0]], rows.at[0], sem0)

        def body(i, c2):
            k0 = i * 2
            pltpu.async_copy(u_hbm.at[sidx.at[k0 + 1]], rows.at[1], sem1)
            pltpu.make_async_copy(
                u_hbm.at[sidx.at[k0]], rows.at[0], sem0).wait()
            pltpu.sync_copy(rows.at[0], acc_sp.at[didx.at[k0]], add=True)

            @pl.when(k0 + 2 < GRP)
            def _():
                pltpu.async_copy(u_hbm.at[sidx.at[k0 + 2]], rows.at[0], sem0)

            pltpu.make_async_copy(
                u_hbm.at[sidx.at[k0 + 1]], rows.at[1], sem1).wait()
            pltpu.sync_copy(rows.at[1], acc_sp.at[didx.at[k0 + 1]], add=True)
            return c2

        lax.fori_loop(0, GRP // 2, body, 0)
        return carry

    lax.fori_loop(0, NG, gbody, 0)
    plsc.subcore_barrier()
    pltpu.sync_copy(
        acc_sp.at[pl.ds(sid * NPS, NPS)],
        acc_out.at[cid, pl.ds(sid * NPS, NPS)],
    )


@functools.partial(
    pl.kernel,
    out_type=jax.ShapeDtypeStruct((NC, NP), jnp.float32),
    mesh=_mesh,
    scratch_types=[
        pltpu.VMEM((CPT, CH), jnp.int32),
        pltpu.VMEM((CPT, CH), jnp.int32),
        pltpu.VMEM((CH,), jnp.float32),
        pltpu.VMEM_SHARED((NP,), jnp.float32),
        pltpu.SemaphoreType.DMA,
    ],
)
def _agg2_kernel(src2d, dst2d, v_hbm, zeros_np, agg_out,
                 sidx, didx, vals, agg_sp, sem):
    cid = lax.axis_index("c")
    sid = lax.axis_index("s")
    wid = cid * NS + sid
    pltpu.sync_copy(src2d.at[pl.ds(wid * CPT, CPT)], sidx)
    pltpu.sync_copy(dst2d.at[pl.ds(wid * CPT, CPT)], didx)

    @pl.when(sid == 0)
    def _():
        pltpu.sync_copy(zeros_np, agg_sp)

    plsc.subcore_barrier()

    def body(k, carry):
        pltpu.async_copy(v_hbm.at[sidx.at[k]], vals, sem).wait()
        pltpu.sync_copy(vals, agg_sp.at[didx.at[k]], add=True)
        return carry

    lax.fori_loop(0, CPT, body, 0)
    plsc.subcore_barrier()
    pltpu.sync_copy(
        agg_sp.at[pl.ds(sid * NPS, NPS)],
        agg_out.at[cid, pl.ds(sid * NPS, NPS)],
    )


def _mm1_body(x_ref, w_ref, d0_ref, d1_ref, u_ref, dinv_ref):
    h = jnp.dot(x_ref[...], w_ref[...], preferred_element_type=jnp.float32)
    deg = d0_ref[...] + d1_ref[...] + 1.0
    dinv = lax.rsqrt(deg)
    u_ref[...] = h * dinv
    dinv_ref[...] = dinv


_mm1 = pl.pallas_call(
    _mm1_body,
    grid=(TG,),
    in_specs=[
        pl.BlockSpec((BLK, D), lambda i: (i, 0)),
        pl.BlockSpec((D, H), lambda i: (0, 0)),
        pl.BlockSpec((BLK, 1), lambda i: (i, 0)),
        pl.BlockSpec((BLK, 1), lambda i: (i, 0)),
    ],
    out_specs=[
        pl.BlockSpec((BLK, H), lambda i: (i, 0)),
        pl.BlockSpec((BLK, 1), lambda i: (i, 0)),
    ],
    out_shape=[
        jax.ShapeDtypeStruct((NP, H), jnp.float32),
        jax.ShapeDtypeStruct((NP, 1), jnp.float32),
    ],
)


def _mm2_body(a0_ref, a1_ref, u_ref, dinv_ref, w2_ref, b1_ref, v_ref):
    dinv = dinv_ref[...]
    out1 = dinv * (a0_ref[...] + a1_ref[...] + u_ref[...]) + b1_ref[...]
    a = jnp.maximum(out1, 0.0)
    z = jnp.dot(a, w2_ref[...], preferred_element_type=jnp.float32)
    v_ref[...] = z * dinv


_mm2 = pl.pallas_call(
    _mm2_body,
    grid=(TG,),
    in_specs=[
        pl.BlockSpec((BLK, H), lambda i: (i, 0)),
        pl.BlockSpec((BLK, H), lambda i: (i, 0)),
        pl.BlockSpec((BLK, H), lambda i: (i, 0)),
        pl.BlockSpec((BLK, 1), lambda i: (i, 0)),
        pl.BlockSpec((H, 1), lambda i: (0, 0)),
        pl.BlockSpec((1, H), lambda i: (0, 0)),
    ],
    out_specs=pl.BlockSpec((BLK, 1), lambda i: (i, 0)),
    out_shape=jax.ShapeDtypeStruct((NP, 1), jnp.float32),
)


def _fin_body(a0_ref, a1_ref, v_ref, dinv_ref, b2_ref, o_ref):
    o_ref[...] = (
        dinv_ref[...] * (a0_ref[...] + a1_ref[...] + v_ref[...]) + b2_ref[...]
    )


_fin = pl.pallas_call(
    _fin_body,
    grid=(TG,),
    in_specs=[
        pl.BlockSpec((BLK, 1), lambda i: (i, 0)),
        pl.BlockSpec((BLK, 1), lambda i: (i, 0)),
        pl.BlockSpec((BLK, 1), lambda i: (i, 0)),
        pl.BlockSpec((BLK, 1), lambda i: (i, 0)),
        pl.BlockSpec((1, 1), lambda i: (0, 0)),
    ],
    out_specs=pl.BlockSpec((BLK, 1), lambda i: (i, 0)),
    out_shape=jax.ShapeDtypeStruct((NP, 1), jnp.float32),
)


def kernel(x, edge_index, W1, b1, W2, b2):
    e = edge_index.shape[1]
    src = edge_index[0].astype(jnp.int32)
    dst = edge_index[1].astype(jnp.int32)
    src_p = jnp.concatenate([src, jnp.zeros((EP - e,), jnp.int32)])
    dst_p = jnp.concatenate([dst, jnp.full((EP - e,), TRASH, jnp.int32)])
    src2d = src_p.reshape(NCH, CH)
    dst2d = dst_p.reshape(NCH, CH)
    zeros_np = jnp.zeros((NP,), jnp.float32)
    zeros_nph = jnp.zeros((NP, H), jnp.float32)
    x_p = jnp.concatenate(
        [x.astype(jnp.float32), jnp.zeros((NP - N, D), jnp.float32)]
    )

    degp = _deg_kernel(dst2d, zeros_np)
    deg0 = degp[0].reshape(NP, 1)
    deg1 = degp[1].reshape(NP, 1)
    u, dinv = _mm1(x_p, W1, deg0, deg1)
    accp = _agg_kernel(src2d, dst2d, u, zeros_nph)
    v = _mm2(accp[0], accp[1], u, dinv, W2, b1.reshape(1, H))
    aggp = _agg2_kernel(src2d, dst2d, v.reshape(NP), zeros_np)
    out = _fin(
        aggp[0].reshape(NP, 1), aggp[1].reshape(NP, 1), v, dinv,
        b2.reshape(1, 1),
    )
    return out[:N]


# speedup vs baseline: 15.3472x; 1.0026x over previous
import functools

import jax
import jax.numpy as jnp
from jax import lax
from jax.experimental import pallas as pl
from jax.experimental.pallas import tpu as pltpu
from jax.experimental.pallas import tpu_sc as plsc

N = 10000
D = 128
H = 128
NP = 10240
TRASH = 10000
NC = 2
NS = 16
L = 16
NW = NC * NS
CH = 128
CPT = 80
EP = NW * CPT * CH
NCH = EP // CH
BLK = 1024
TG = NP // BLK
NPS = NP // NS

_mesh = plsc.VectorSubcoreMesh(
    core_axis_name="c", subcore_axis_name="s", num_cores=NC, num_subcores=NS
)


@functools.partial(
    pl.kernel,
    out_type=jax.ShapeDtypeStruct((NC, NP), jnp.float32),
    mesh=_mesh,
    scratch_types=[
        pltpu.VMEM((CPT, CH), jnp.int32),
        pltpu.VMEM((CH,), jnp.float32),
        pltpu.VMEM_SHARED((NP,), jnp.float32),
    ],
)
def _deg_kernel(dst2d, zeros_np, deg_out, didx, ones, deg_sp):
    cid = lax.axis_index("c")
    sid = lax.axis_index("s")
    wid = cid * NS + sid
    for j in range(CH // L):
        ones[pl.ds(j * L, L)] = jnp.ones((L,), jnp.float32)
    pltpu.sync_copy(dst2d.at[pl.ds(wid * CPT, CPT)], didx)

    @pl.when(sid == 0)
    def _():
        pltpu.sync_copy(zeros_np, deg_sp)

    plsc.subcore_barrier()

    def body(k, carry):
        pltpu.sync_copy(ones, deg_sp.at[didx.at[k]], add=True)
        return carry

    lax.fori_loop(0, CPT, body, 0)
    plsc.subcore_barrier()
    pltpu.sync_copy(
        deg_sp.at[pl.ds(sid * NPS, NPS)],
        deg_out.at[cid, pl.ds(sid * NPS, NPS)],
    )


GRP = 16
NG = CPT // GRP


@functools.partial(
    pl.kernel,
    out_type=jax.ShapeDtypeStruct((NC, NP, H), jnp.float32),
    mesh=_mesh,
    scratch_types=[
        pltpu.VMEM((GRP, CH), jnp.int32),
        pltpu.VMEM((GRP, CH), jnp.int32),
        pltpu.VMEM((2, CH, H), jnp.float32),
        pltpu.VMEM_SHARED((NP, H), jnp.float32),
        pltpu.SemaphoreType.DMA,
        pltpu.SemaphoreType.DMA,
    ],
)
def _agg_kernel(src2d, dst2d, u_hbm, zeros_nph, acc_out,
                sidx, didx, rows, acc_sp, sem0, sem1):
    cid = lax.axis_index("c")
    sid = lax.axis_index("s")
    wid = cid * NS + sid

    @pl.when(sid == 0)
    def _():
        pltpu.sync_copy(zeros_nph, acc_sp)

    plsc.subcore_barrier()

    def gbody(g, carry):
        base = wid * CPT + g * GRP
        pltpu.sync_copy(src2d.at[pl.ds(base, GRP)], sidx)
        pltpu.sync_copy(dst2d.at[pl.ds(base, GRP)], didx)
        pltpu.async_copy(u_hbm.at[sidx.at[0]], rows.at[0], sem0)

        def body(i, c2):
            k0 = i * 2
            pltpu.async_copy(u_hbm.at[sidx.at[k0 + 1]], rows.at[1], sem1)
            pltpu.make_async_copy(
                u_hbm.at[sidx.at[k0]], rows.at[0], sem0).wait()
            pltpu.sync_copy(rows.at[0], acc_sp.at[didx.at[k0]], add=True)

            @pl.when(k0 + 2 < GRP)
            def _():
                pltpu.async_copy(u_hbm.at[sidx.at[k0 + 2]], rows.at[0], sem0)

            pltpu.make_async_copy(
                u_hbm.at[sidx.at[k0 + 1]], rows.at[1], sem1).wait()
            pltpu.sync_copy(rows.at[1], acc_sp.at[didx.at[k0 + 1]], add=True)
            return c2

        lax.fori_loop(0, GRP // 2, body, 0)
        return carry

    lax.fori_loop(0, NG, gbody, 0)
    plsc.subcore_barrier()
    pltpu.sync_copy(
        acc_sp.at[pl.ds(sid * NPS, NPS)],
        acc_out.at[cid, pl.ds(sid * NPS, NPS)],
    )


@functools.partial(
    pl.kernel,
    out_type=jax.ShapeDtypeStruct((NC, NP), jnp.float32),
    mesh=_mesh,
    scratch_types=[
        pltpu.VMEM((CPT, CH), jnp.int32),
        pltpu.VMEM((CPT, CH), jnp.int32),
        pltpu.VMEM((CH,), jnp.float32),
        pltpu.VMEM_SHARED((NP,), jnp.float32),
        pltpu.SemaphoreType.DMA,
    ],
)
def _agg2_kernel(src2d, dst2d, v_hbm, zeros_np, agg_out,
                 sidx, didx, vals, agg_sp, sem):
    cid = lax.axis_index("c")
    sid = lax.axis_index("s")
    wid = cid * NS + sid
    pltpu.sync_copy(src2d.at[pl.ds(wid * CPT, CPT)], sidx)
    pltpu.sync_copy(dst2d.at[pl.ds(wid * CPT, CPT)], didx)

    @pl.when(sid == 0)
    def _():
        pltpu.sync_copy(zeros_np, agg_sp)

    plsc.subcore_barrier()

    def body(k, carry):
        pltpu.async_copy(v_hbm.at[sidx.at[k]], vals, sem).wait()
        pltpu.sync_copy(vals, agg_sp.at[didx.at[k]], add=True)
        return carry

    lax.fori_loop(0, CPT, body, 0)
    plsc.subcore_barrier()
    pltpu.sync_copy(
        agg_sp.at[pl.ds(sid * NPS, NPS)],
        agg_out.at[cid, pl.ds(sid * NPS, NPS)],
    )


def _mm1_body(x_ref, w_ref, d0_ref, d1_ref, u_ref, dinv_ref):
    h = jnp.dot(x_ref[...], w_ref[...], preferred_element_type=jnp.float32)
    deg = d0_ref[...] + d1_ref[...] + 1.0
    dinv = lax.rsqrt(deg)
    u_ref[...] = h * dinv
    dinv_ref[...] = dinv


_mm1 = pl.pallas_call(
    _mm1_body,
    grid=(TG,),
    in_specs=[
        pl.BlockSpec((BLK, D), lambda i: (i, 0)),
        pl.BlockSpec((D, H), lambda i: (0, 0)),
        pl.BlockSpec((BLK, 1), lambda i: (i, 0)),
        pl.BlockSpec((BLK, 1), lambda i: (i, 0)),
    ],
    out_specs=[
        pl.BlockSpec((BLK, H), lambda i: (i, 0)),
        pl.BlockSpec((BLK, 1), lambda i: (i, 0)),
    ],
    out_shape=[
        jax.ShapeDtypeStruct((NP, H), jnp.float32),
        jax.ShapeDtypeStruct((NP, 1), jnp.float32),
    ],
)


def _mm2_body(a0_ref, a1_ref, u_ref, dinv_ref, w2_ref, b1_ref, v_ref):
    dinv = dinv_ref[...]
    out1 = dinv * (a0_ref[...] + a1_ref[...] + u_ref[...]) + b1_ref[...]
    a = jnp.maximum(out1, 0.0)
    z = jnp.dot(a, w2_ref[...], preferred_element_type=jnp.float32)
    v_ref[...] = z * dinv


_mm2 = pl.pallas_call(
    _mm2_body,
    grid=(TG,),
    in_specs=[
        pl.BlockSpec((BLK, H), lambda i: (i, 0)),
        pl.BlockSpec((BLK, H), lambda i: (i, 0)),
        pl.BlockSpec((BLK, H), lambda i: (i, 0)),
        pl.BlockSpec((BLK, 1), lambda i: (i, 0)),
        pl.BlockSpec((H, 1), lambda i: (0, 0)),
        pl.BlockSpec((1, H), lambda i: (0, 0)),
    ],
    out_specs=pl.BlockSpec((BLK, 1), lambda i: (i, 0)),
    out_shape=jax.ShapeDtypeStruct((NP, 1), jnp.float32),
)


def _fin_body(a0_ref, a1_ref, v_ref, dinv_ref, b2_ref, o_ref):
    o_ref[...] = (
        dinv_ref[...] * (a0_ref[...] + a1_ref[...] + v_ref[...]) + b2_ref[...]
    )


_fin = pl.pallas_call(
    _fin_body,
    grid=(TG,),
    in_specs=[
        pl.BlockSpec((BLK, 1), lambda i: (i, 0)),
        pl.BlockSpec((BLK, 1), lambda i: (i, 0)),
        pl.BlockSpec((BLK, 1), lambda i: (i, 0)),
        pl.BlockSpec((BLK, 1), lambda i: (i, 0)),
        pl.BlockSpec((1, 1), lambda i: (0, 0)),
    ],
    out_specs=pl.BlockSpec((BLK, 1), lambda i: (i, 0)),
    out_shape=jax.ShapeDtypeStruct((NP, 1), jnp.float32),
)


def kernel(x, edge_index, W1, b1, W2, b2):
    e = edge_index.shape[1]
    src = edge_index[0].astype(jnp.int32)
    dst = edge_index[1].astype(jnp.int32)
    src_p = jnp.concatenate([src, jnp.zeros((EP - e,), jnp.int32)])
    pad_dst = TRASH + jax.lax.rem(
        jnp.arange(EP - e, dtype=jnp.int32), jnp.int32(NP - N)
    )
    dst_p = jnp.concatenate([dst, pad_dst])
    src2d = src_p.reshape(NCH, CH)
    dst2d = dst_p.reshape(NCH, CH)
    zeros_np = jnp.zeros((NP,), jnp.float32)
    zeros_nph = jnp.zeros((NP, H), jnp.float32)
    x_p = jnp.concatenate(
        [x.astype(jnp.float32), jnp.zeros((NP - N, D), jnp.float32)]
    )

    degp = _deg_kernel(dst2d, zeros_np)
    deg0 = degp[0].reshape(NP, 1)
    deg1 = degp[1].reshape(NP, 1)
    u, dinv = _mm1(x_p, W1, deg0, deg1)
    accp = _agg_kernel(src2d, dst2d, u, zeros_nph)
    v = _mm2(accp[0], accp[1], u, dinv, W2, b1.reshape(1, H))
    aggp = _agg2_kernel(src2d, dst2d, v.reshape(NP), zeros_np)
    out = _fin(
        aggp[0].reshape(NP, 1), aggp[1].reshape(NP, 1), v, dinv,
        b2.reshape(1, 1),
    )
    return out[:N]


# speedup vs baseline: 17.2313x; 1.1228x over previous
import functools

import jax
import jax.numpy as jnp
from jax import lax
from jax.experimental import pallas as pl
from jax.experimental.pallas import tpu as pltpu
from jax.experimental.pallas import tpu_sc as plsc

N = 10000
D = 128
H = 128
NP = 10240
TRASH = 10000
NC = 2
NS = 16
L = 16
NW = NC * NS
CH = 128
CPT = 80
EP = NW * CPT * CH
NCH = EP // CH
BLK = 1024
TG = NP // BLK
NPS = NP // NS

_mesh = plsc.VectorSubcoreMesh(
    core_axis_name="c", subcore_axis_name="s", num_cores=NC, num_subcores=NS
)


@functools.partial(
    pl.kernel,
    out_type=jax.ShapeDtypeStruct((NC, NP), jnp.float32),
    mesh=_mesh,
    scratch_types=[
        pltpu.VMEM((CPT, CH), jnp.int32),
        pltpu.VMEM((CH,), jnp.float32),
        pltpu.VMEM_SHARED((NP,), jnp.float32),
    ],
)
def _deg_kernel(dst2d, zeros_np, deg_out, didx, ones, deg_sp):
    cid = lax.axis_index("c")
    sid = lax.axis_index("s")
    wid = cid * NS + sid
    for j in range(CH // L):
        ones[pl.ds(j * L, L)] = jnp.ones((L,), jnp.float32)
    pltpu.sync_copy(dst2d.at[pl.ds(wid * CPT, CPT)], didx)

    @pl.when(sid == 0)
    def _():
        pltpu.sync_copy(zeros_np, deg_sp)

    plsc.subcore_barrier()

    def body(k, carry):
        pltpu.sync_copy(ones, deg_sp.at[didx.at[k]], add=True)
        return carry

    lax.fori_loop(0, CPT, body, 0)
    plsc.subcore_barrier()
    pltpu.sync_copy(
        deg_sp.at[pl.ds(sid * NPS, NPS)],
        deg_out.at[cid, pl.ds(sid * NPS, NPS)],
    )


GRP = 16
NG0 = 8
NG1 = 2
SPLIT = NS * NG0 * GRP


@functools.partial(
    pl.kernel,
    out_type=jax.ShapeDtypeStruct((NC, NP, H), jnp.float32),
    mesh=_mesh,
    scratch_types=[
        pltpu.VMEM((GRP, CH), jnp.int32),
        pltpu.VMEM((GRP, CH), jnp.int32),
        pltpu.VMEM((2, CH, H), jnp.float32),
        pltpu.VMEM_SHARED((NP, H), jnp.float32),
        pltpu.SemaphoreType.DMA,
        pltpu.SemaphoreType.DMA,
    ],
)
def _agg_kernel(src2d, dst2d, u_hbm, zeros_nph, acc_out,
                sidx, didx, rows, acc_sp, sem0, sem1):
    cid = lax.axis_index("c")
    sid = lax.axis_index("s")
    ng = jnp.where(cid == 0, NG0, NG1)
    start = jnp.where(
        cid == 0, sid * (NG0 * GRP), SPLIT + sid * (NG1 * GRP)
    )

    @pl.when(sid == 0)
    def _():
        pltpu.sync_copy(zeros_nph, acc_sp)

    plsc.subcore_barrier()

    def gbody(g, carry):
        base = start + g * GRP
        pltpu.sync_copy(src2d.at[pl.ds(base, GRP)], sidx)
        pltpu.sync_copy(dst2d.at[pl.ds(base, GRP)], didx)
        pltpu.async_copy(u_hbm.at[sidx.at[0]], rows.at[0], sem0)

        def body(i, c2):
            k0 = i * 2
            pltpu.async_copy(u_hbm.at[sidx.at[k0 + 1]], rows.at[1], sem1)
            pltpu.make_async_copy(
                u_hbm.at[sidx.at[k0]], rows.at[0], sem0).wait()
            pltpu.sync_copy(rows.at[0], acc_sp.at[didx.at[k0]], add=True)

            @pl.when(k0 + 2 < GRP)
            def _():
                pltpu.async_copy(u_hbm.at[sidx.at[k0 + 2]], rows.at[0], sem0)

            pltpu.make_async_copy(
                u_hbm.at[sidx.at[k0 + 1]], rows.at[1], sem1).wait()
            pltpu.sync_copy(rows.at[1], acc_sp.at[didx.at[k0 + 1]], add=True)
            return c2

        lax.fori_loop(0, GRP // 2, body, 0)
        return carry

    lax.fori_loop(0, ng, gbody, 0)
    plsc.subcore_barrier()
    pltpu.sync_copy(
        acc_sp.at[pl.ds(sid * NPS, NPS)],
        acc_out.at[cid, pl.ds(sid * NPS, NPS)],
    )


@functools.partial(
    pl.kernel,
    out_type=jax.ShapeDtypeStruct((NC, NP), jnp.float32),
    mesh=_mesh,
    scratch_types=[
        pltpu.VMEM((CPT, CH), jnp.int32),
        pltpu.VMEM((CPT, CH), jnp.int32),
        pltpu.VMEM((2, CH), jnp.float32),
        pltpu.VMEM_SHARED((NP,), jnp.float32),
        pltpu.SemaphoreType.DMA,
        pltpu.SemaphoreType.DMA,
    ],
)
def _agg2_kernel(src2d, dst2d, v_hbm, zeros_np, agg_out,
                 sidx, didx, vals, agg_sp, sem0, sem1):
    cid = lax.axis_index("c")
    sid = lax.axis_index("s")
    wid = cid * NS + sid
    pltpu.sync_copy(src2d.at[pl.ds(wid * CPT, CPT)], sidx)
    pltpu.sync_copy(dst2d.at[pl.ds(wid * CPT, CPT)], didx)

    @pl.when(sid == 0)
    def _():
        pltpu.sync_copy(zeros_np, agg_sp)

    plsc.subcore_barrier()
    pltpu.async_copy(v_hbm.at[sidx.at[0]], vals.at[0], sem0)

    def body(i, carry):
        k0 = i * 2
        pltpu.async_copy(v_hbm.at[sidx.at[k0 + 1]], vals.at[1], sem1)
        pltpu.make_async_copy(
            v_hbm.at[sidx.at[k0]], vals.at[0], sem0).wait()
        pltpu.sync_copy(vals.at[0], agg_sp.at[didx.at[k0]], add=True)

        @pl.when(k0 + 2 < CPT)
        def _():
            pltpu.async_copy(v_hbm.at[sidx.at[k0 + 2]], vals.at[0], sem0)

        pltpu.make_async_copy(
            v_hbm.at[sidx.at[k0 + 1]], vals.at[1], sem1).wait()
        pltpu.sync_copy(vals.at[1], agg_sp.at[didx.at[k0 + 1]], add=True)
        return carry

    lax.fori_loop(0, CPT // 2, body, 0)
    plsc.subcore_barrier()
    pltpu.sync_copy(
        agg_sp.at[pl.ds(sid * NPS, NPS)],
        agg_out.at[cid, pl.ds(sid * NPS, NPS)],
    )


def _mm1_body(x_ref, w_ref, d0_ref, d1_ref, u_ref, dinv_ref):
    h = jnp.dot(x_ref[...], w_ref[...], preferred_element_type=jnp.float32)
    deg = d0_ref[...] + d1_ref[...] + 1.0
    dinv = lax.rsqrt(deg)
    u_ref[...] = h * dinv
    dinv_ref[...] = dinv


_mm1 = pl.pallas_call(
    _mm1_body,
    grid=(TG,),
    in_specs=[
        pl.BlockSpec((BLK, D), lambda i: (i, 0)),
        pl.BlockSpec((D, H), lambda i: (0, 0)),
        pl.BlockSpec((BLK, 1), lambda i: (i, 0)),
        pl.BlockSpec((BLK, 1), lambda i: (i, 0)),
    ],
    out_specs=[
        pl.BlockSpec((BLK, H), lambda i: (i, 0)),
        pl.BlockSpec((BLK, 1), lambda i: (i, 0)),
    ],
    out_shape=[
        jax.ShapeDtypeStruct((NP, H), jnp.float32),
        jax.ShapeDtypeStruct((NP, 1), jnp.float32),
    ],
)


def _mm2_body(a0_ref, a1_ref, u_ref, dinv_ref, w2_ref, b1_ref, v_ref):
    dinv = dinv_ref[...]
    out1 = dinv * (a0_ref[...] + a1_ref[...] + u_ref[...]) + b1_ref[...]
    a = jnp.maximum(out1, 0.0)
    z = jnp.dot(a, w2_ref[...], preferred_element_type=jnp.float32)
    v_ref[...] = z * dinv


_mm2 = pl.pallas_call(
    _mm2_body,
    grid=(TG,),
    in_specs=[
        pl.BlockSpec((BLK, H), lambda i: (i, 0)),
        pl.BlockSpec((BLK, H), lambda i: (i, 0)),
        pl.BlockSpec((BLK, H), lambda i: (i, 0)),
        pl.BlockSpec((BLK, 1), lambda i: (i, 0)),
        pl.BlockSpec((H, 1), lambda i: (0, 0)),
        pl.BlockSpec((1, H), lambda i: (0, 0)),
    ],
    out_specs=pl.BlockSpec((BLK, 1), lambda i: (i, 0)),
    out_shape=jax.ShapeDtypeStruct((NP, 1), jnp.float32),
)


def _fin_body(a0_ref, a1_ref, v_ref, dinv_ref, b2_ref, o_ref):
    o_ref[...] = (
        dinv_ref[...] * (a0_ref[...] + a1_ref[...] + v_ref[...]) + b2_ref[...]
    )


_fin = pl.pallas_call(
    _fin_body,
    grid=(TG,),
    in_specs=[
        pl.BlockSpec((BLK, 1), lambda i: (i, 0)),
        pl.BlockSpec((BLK, 1), lambda i: (i, 0)),
        pl.BlockSpec((BLK, 1), lambda i: (i, 0)),
        pl.BlockSpec((BLK, 1), lambda i: (i, 0)),
        pl.BlockSpec((1, 1), lambda i: (0, 0)),
    ],
    out_specs=pl.BlockSpec((BLK, 1), lambda i: (i, 0)),
    out_shape=jax.ShapeDtypeStruct((NP, 1), jnp.float32),
)


def kernel(x, edge_index, W1, b1, W2, b2):
    e = edge_index.shape[1]
    src = edge_index[0].astype(jnp.int32)
    dst = edge_index[1].astype(jnp.int32)
    src_p = jnp.concatenate([src, jnp.zeros((EP - e,), jnp.int32)])
    pad_dst = TRASH + jax.lax.rem(
        jnp.arange(EP - e, dtype=jnp.int32), jnp.int32(NP - N)
    )
    dst_p = jnp.concatenate([dst, pad_dst])
    src2d = src_p.reshape(NCH, CH)
    dst2d = dst_p.reshape(NCH, CH)
    zeros_np = jnp.zeros((NP,), jnp.float32)
    zeros_nph = jnp.zeros((NP, H), jnp.float32)
    x_p = jnp.concatenate(
        [x.astype(jnp.float32), jnp.zeros((NP - N, D), jnp.float32)]
    )

    degp = _deg_kernel(dst2d, zeros_np)
    deg0 = degp[0].reshape(NP, 1)
    deg1 = degp[1].reshape(NP, 1)
    u, dinv = _mm1(x_p, W1, deg0, deg1)
    accp = _agg_kernel(src2d, dst2d, u, zeros_nph)
    v = _mm2(accp[0], accp[1], u, dinv, W2, b1.reshape(1, H))
    aggp = _agg2_kernel(src2d, dst2d, v.reshape(NP), zeros_np)
    out = _fin(
        aggp[0].reshape(NP, 1), aggp[1].reshape(NP, 1), v, dinv,
        b2.reshape(1, 1),
    )
    return out[:N]
